# baseline (speedup 1.0000x reference)
import functools

import jax
import jax.numpy as jnp
from jax import lax
from jax.experimental import pallas as pl
from jax.experimental.pallas import tpu as pltpu
from jax.experimental.pallas import tpu_sc as plsc

_PARAMS = pltpu.CompilerParams(
    use_tc_tiling_on_sc=True, needs_layout_passes=False)


@functools.lru_cache(maxsize=None)
def _make_repack(d, v):
    info = plsc.get_sparse_core_info()
    nw = info.num_cores * info.num_subcores
    lanes = 128
    n_full = v // lanes
    rem = v - n_full * lanes

    mesh = plsc.VectorSubcoreMesh(core_axis_name="c", subcore_axis_name="s")

    @functools.partial(
        pl.kernel,
        mesh=mesh,
        out_type=jax.ShapeDtypeStruct((v // 2, 2 * d), jnp.float32),
        compiler_params=_PARAMS,
        scratch_types=[
            pltpu.VMEM((2, d, lanes), jnp.float32),
            pltpu.VMEM((2, d, lanes), jnp.float32),
            pltpu.VMEM((d, rem), jnp.float32),
            pltpu.VMEM((rem // 2, 2 * d), jnp.float32),
            pltpu.SemaphoreType.DMA,
            pltpu.SemaphoreType.DMA,
        ],
    )
    def repack(et_hbm, tabp_hbm, vbuf, obuf, vtail, otail, gsem, wsem):
        w = lax.axis_index("s") * info.num_cores + lax.axis_index("c")
        iota = lax.iota(jnp.int32, 16)
        nblk = (n_full - w + nw - 1) // nw

        def rd(c, slot):
            return pltpu.make_async_copy(
                et_hbm.at[:, pl.ds(c * lanes, lanes)], vbuf.at[slot], gsem)

        def wr(c, slot):
            return pltpu.make_async_copy(
                obuf.at[slot], tabp_hbm.at[pl.ds(c * (lanes // 2), lanes // 2), :],
                wsem)

        def transpose(slot, vb, ob, npair):
            for kg in range(2 * d // 16):
                row_v = (16 * kg) % d + iota
                off = 1 if 16 * kg >= d else 0
                for p in range(npair):
                    vals = plsc.load_gather(
                        vb, [row_v, jnp.full((16,), 2 * p + off, jnp.int32)])
                    ob[p, pl.ds(16 * kg, 16)] = vals

        @pl.when(nblk > 0)
        def _():
            rd(w, 0).start()

            def body(j, carry):
                c = w + j * nw
                slot = lax.rem(j, 2)

                @pl.when(slot == 0)
                def _():
                    rd(c, 0).wait()

                    @pl.when(j >= 2)
                    def _():
                        wr(c - 2 * nw, 0).wait()

                    transpose(0, vbuf.at[0], obuf.at[0], d)
                    wr(c, 0).start()

                    @pl.when(j + 1 < nblk)
                    def _():
                        rd(c + nw, 1).start()

                @pl.when(slot == 1)
                def _():
                    rd(c, 1).wait()

                    @pl.when(j >= 2)
                    def _():
                        wr(c - 2 * nw, 1).wait()

                    transpose(1, vbuf.at[1], obuf.at[1], d)
                    wr(c, 1).start()

                    @pl.when(j + 1 < nblk)
                    def _():
                        rd(c + nw, 0).start()

                return carry

            lax.fori_loop(0, nblk, body, 0)
            @pl.when(nblk >= 2)
            def _():
                wr(0, 0).wait()
            wr(0, 0).wait()

        @pl.when(w == nw - 1)
        def _():
            pltpu.sync_copy(et_hbm.at[:, pl.ds(n_full * lanes, rem)], vtail)
            for kg in range(2 * d // 16):
                row_v = (16 * kg) % d + iota
                off = 1 if 16 * kg >= d else 0
                for p in range(rem // 2):
                    vals = plsc.load_gather(
                        vtail, [row_v, jnp.full((16,), 2 * p + off, jnp.int32)])
                    otail[p, pl.ds(16 * kg, 16)] = vals
            pltpu.sync_copy(
                otail, tabp_hbm.at[pl.ds(n_full * (lanes // 2), rem // 2), :])

    return repack


@functools.lru_cache(maxsize=None)
def _make_lookup(t_dim, b_dim, vh, d):
    info = plsc.get_sparse_core_info()
    nw = info.num_cores * info.num_subcores
    lanes = 128
    assert b_dim == nw * lanes
    d2 = 2 * d

    mesh = plsc.VectorSubcoreMesh(core_axis_name="c", subcore_axis_name="s")

    @functools.partial(
        pl.kernel,
        mesh=mesh,
        out_type=jax.ShapeDtypeStruct((t_dim, d, b_dim), jnp.float32),
        compiler_params=_PARAMS,
        scratch_types=[
            pltpu.VMEM((t_dim, lanes), jnp.int32),
            pltpu.VMEM((2, lanes), jnp.int32),
            pltpu.VMEM((2, lanes, d2), jnp.float32),
            pltpu.VMEM((2, d, lanes), jnp.float32),
            pltpu.SemaphoreType.DMA,
            pltpu.SemaphoreType.DMA,
        ],
    )
    def lookup(xt_hbm, tab_hbm, out_hbm, xtcol, hbuf, pairbuf, staging,
               gsem, wsem):
        w = lax.axis_index("s") * info.num_cores + lax.axis_index("c")
        col = w * lanes

        pltpu.sync_copy(xt_hbm.at[:, pl.ds(col, lanes)], xtcol)

        iota = lax.iota(jnp.int32, 16)

        def make_h(t, slot):
            for k in range(lanes // 16):
                v = xtcol[t, pl.ds(16 * k, 16)]
                hbuf[slot, pl.ds(16 * k, 16)] = lax.shift_right_logical(v, 1)

        def gather(slot):
            return pltpu.make_async_copy(
                tab_hbm.at[hbuf.at[slot]], pairbuf.at[slot], gsem)

        def wb(t, slot):
            return pltpu.make_async_copy(
                staging.at[slot], out_hbm.at[t, :, pl.ds(col, lanes)], wsem)

        def transpose(t, slot):
            for k in range(lanes // 16):
                row_v = 16 * k + iota
                idx_v = xtcol[t, pl.ds(16 * k, 16)]
                off_v = lax.shift_left(
                    lax.bitwise_and(idx_v, jnp.int32(1)), 6)
                for dd in range(d):
                    vals = plsc.load_gather(
                        pairbuf.at[slot], [row_v, off_v + dd])
                    staging[slot, dd, pl.ds(16 * k, 16)] = vals

        make_h(0, 0)
        gather(0).start()
        make_h(1, 1)
        gather(1).start()

        def body(i, carry):
            for par in range(2):
                t = 2 * i + par
                gather(par).wait()

                @pl.when(t >= 2)
                def _():
                    wb(t - 2, par).wait()

                transpose(t, par)
                wb(t, par).start()

                @pl.when(t + 2 < t_dim)
                def _():
                    make_h(t + 2, par)
                    gather(par).start()
            return carry

        lax.fori_loop(0, t_dim // 2, body, 0)
        wb(t_dim - 2, 0).wait()
        wb(t_dim - 1, 1).wait()

    return lookup


def kernel(x, embedding):
    b, t = x.shape
    v, d = embedding.shape
    xt = x.astype(jnp.int32).T
    et = embedding.T
    tabp = _make_repack(d, v)(et)
    out_p = _make_lookup(t, b, v // 2, d)(xt, tabp)
    return jnp.transpose(out_p, (2, 0, 1))


# speedup vs baseline: 1.8005x; 1.8005x over previous
import functools

import jax
import jax.numpy as jnp
from jax import lax
from jax.experimental import pallas as pl
from jax.experimental.pallas import tpu as pltpu
from jax.experimental.pallas import tpu_sc as plsc

_PARAMS = pltpu.CompilerParams(
    use_tc_tiling_on_sc=True, needs_layout_passes=False)


@functools.lru_cache(maxsize=None)
def _make_repack(d, v):
    info = plsc.get_sparse_core_info()
    nw = info.num_cores * info.num_subcores
    lanes = 128
    n_full = v // lanes
    rem = v - n_full * lanes

    mesh = plsc.VectorSubcoreMesh(core_axis_name="c", subcore_axis_name="s")

    @functools.partial(
        pl.kernel,
        mesh=mesh,
        out_type=jax.ShapeDtypeStruct((v // 2, 2 * d), jnp.float32),
        compiler_params=_PARAMS,
        scratch_types=[
            pltpu.VMEM((2, d, lanes), jnp.float32),
            pltpu.VMEM((2, d, lanes), jnp.float32),
            pltpu.VMEM((d, rem), jnp.float32),
            pltpu.VMEM((rem // 2, 2 * d), jnp.float32),
            pltpu.SemaphoreType.DMA,
            pltpu.SemaphoreType.DMA,
        ],
    )
    def repack(et_hbm, tabp_hbm, vbuf, obuf, vtail, otail, gsem, wsem):
        w = lax.axis_index("s") * info.num_cores + lax.axis_index("c")
        iota = lax.iota(jnp.int32, 16)
        nblk = (n_full - w + nw - 1) // nw

        def rd(c, slot):
            return pltpu.make_async_copy(
                et_hbm.at[:, pl.ds(c * lanes, lanes)], vbuf.at[slot], gsem)

        def wr(c, slot):
            return pltpu.make_async_copy(
                obuf.at[slot], tabp_hbm.at[pl.ds(c * (lanes // 2), lanes // 2), :],
                wsem)

        def transpose(slot, vb, ob, npair):
            @plsc.parallel_loop(0, npair, 1, unroll=8)
            def _(p):
                for kg in range(2 * d // 16):
                    row_v = (16 * kg) % d + iota
                    off = 1 if 16 * kg >= d else 0
                    col_v = jnp.full((16,), off, jnp.int32) + 2 * p
                    vals = plsc.load_gather(vb, [row_v, col_v])
                    ob[p, pl.ds(16 * kg, 16)] = vals

        @pl.when(nblk > 0)
        def _():
            rd(w, 0).start()

            def body(j, carry):
                c = w + j * nw
                slot = lax.rem(j, 2)

                @pl.when(slot == 0)
                def _():
                    rd(c, 0).wait()

                    @pl.when(j >= 2)
                    def _():
                        wr(c - 2 * nw, 0).wait()

                    transpose(0, vbuf.at[0], obuf.at[0], d)
                    wr(c, 0).start()

                    @pl.when(j + 1 < nblk)
                    def _():
                        rd(c + nw, 1).start()

                @pl.when(slot == 1)
                def _():
                    rd(c, 1).wait()

                    @pl.when(j >= 2)
                    def _():
                        wr(c - 2 * nw, 1).wait()

                    transpose(1, vbuf.at[1], obuf.at[1], d)
                    wr(c, 1).start()

                    @pl.when(j + 1 < nblk)
                    def _():
                        rd(c + nw, 0).start()

                return carry

            lax.fori_loop(0, nblk, body, 0)
            @pl.when(nblk >= 2)
            def _():
                wr(0, 0).wait()
            wr(0, 0).wait()

        @pl.when(w == nw - 1)
        def _():
            pltpu.sync_copy(et_hbm.at[:, pl.ds(n_full * lanes, rem)], vtail)

            @plsc.parallel_loop(0, rem // 2, 1, unroll=8)
            def _(p):
                for kg in range(2 * d // 16):
                    row_v = (16 * kg) % d + iota
                    off = 1 if 16 * kg >= d else 0
                    col_v = jnp.full((16,), off, jnp.int32) + 2 * p
                    vals = plsc.load_gather(vtail, [row_v, col_v])
                    otail[p, pl.ds(16 * kg, 16)] = vals
            pltpu.sync_copy(
                otail, tabp_hbm.at[pl.ds(n_full * (lanes // 2), rem // 2), :])

    return repack


@functools.lru_cache(maxsize=None)
def _make_lookup(t_dim, b_dim, vh, d):
    info = plsc.get_sparse_core_info()
    nw = info.num_cores * info.num_subcores
    lanes = 128
    assert b_dim == nw * lanes
    d2 = 2 * d

    mesh = plsc.VectorSubcoreMesh(core_axis_name="c", subcore_axis_name="s")

    @functools.partial(
        pl.kernel,
        mesh=mesh,
        out_type=jax.ShapeDtypeStruct((t_dim, d, b_dim), jnp.float32),
        compiler_params=_PARAMS,
        scratch_types=[
            pltpu.VMEM((t_dim, lanes), jnp.int32),
            pltpu.VMEM((2, lanes), jnp.int32),
            pltpu.VMEM((2, lanes, d2), jnp.float32),
            pltpu.VMEM((2, d, lanes), jnp.float32),
            pltpu.SemaphoreType.DMA,
            pltpu.SemaphoreType.DMA,
        ],
    )
    def lookup(xt_hbm, tab_hbm, out_hbm, xtcol, hbuf, pairbuf, staging,
               gsem, wsem):
        w = lax.axis_index("s") * info.num_cores + lax.axis_index("c")
        col = w * lanes

        pltpu.sync_copy(xt_hbm.at[:, pl.ds(col, lanes)], xtcol)

        iota = lax.iota(jnp.int32, 16)

        def make_h(t, slot):
            for k in range(lanes // 16):
                v = xtcol[t, pl.ds(16 * k, 16)]
                hbuf[slot, pl.ds(16 * k, 16)] = lax.shift_right_logical(v, 1)

        def gather(slot):
            return pltpu.make_async_copy(
                tab_hbm.at[hbuf.at[slot]], pairbuf.at[slot], gsem)

        def wb(t, slot):
            return pltpu.make_async_copy(
                staging.at[slot], out_hbm.at[t, :, pl.ds(col, lanes)], wsem)

        def transpose(t, slot):
            for k in range(lanes // 16):
                row_v = 16 * k + iota
                idx_v = xtcol[t, pl.ds(16 * k, 16)]
                off_v = lax.shift_left(
                    lax.bitwise_and(idx_v, jnp.int32(1)), 6)

                @plsc.parallel_loop(0, d, 1, unroll=8)
                def _(dd):
                    vals = plsc.load_gather(
                        pairbuf.at[slot], [row_v, off_v + dd])
                    staging[slot, dd, pl.ds(16 * k, 16)] = vals

        make_h(0, 0)
        gather(0).start()
        make_h(1, 1)
        gather(1).start()

        def body(i, carry):
            for par in range(2):
                t = 2 * i + par
                gather(par).wait()

                @pl.when(t >= 2)
                def _():
                    wb(t - 2, par).wait()

                transpose(t, par)
                wb(t, par).start()

                @pl.when(t + 2 < t_dim)
                def _():
                    make_h(t + 2, par)
                    gather(par).start()
            return carry

        lax.fori_loop(0, t_dim // 2, body, 0)
        wb(t_dim - 2, 0).wait()
        wb(t_dim - 1, 1).wait()

    return lookup


def kernel(x, embedding):
    b, t = x.shape
    v, d = embedding.shape
    xt = x.astype(jnp.int32).T
    et = embedding.T
    tabp = _make_repack(d, v)(et)
    out_p = _make_lookup(t, b, v // 2, d)(xt, tabp)
    return jnp.transpose(out_p, (2, 0, 1))


# speedup vs baseline: 1.8894x; 1.0494x over previous
import functools

import jax
import jax.numpy as jnp
from jax import lax
from jax.experimental import pallas as pl
from jax.experimental.pallas import tpu as pltpu
from jax.experimental.pallas import tpu_sc as plsc

_PARAMS = pltpu.CompilerParams(
    use_tc_tiling_on_sc=True, needs_layout_passes=False)


@functools.lru_cache(maxsize=None)
def _make_repack(d, v):
    info = plsc.get_sparse_core_info()
    nw = info.num_cores * info.num_subcores
    lanes = 128
    n_full = v // lanes
    rem = v - n_full * lanes

    mesh = plsc.VectorSubcoreMesh(core_axis_name="c", subcore_axis_name="s")

    @functools.partial(
        pl.kernel,
        mesh=mesh,
        out_type=jax.ShapeDtypeStruct((v // 2, 2 * d), jnp.float32),
        compiler_params=_PARAMS,
        scratch_types=[
            pltpu.VMEM((2, d, lanes), jnp.float32),
            pltpu.VMEM((2, d, lanes), jnp.float32),
            pltpu.VMEM((d, rem), jnp.float32),
            pltpu.VMEM((rem // 2, 2 * d), jnp.float32),
            pltpu.SemaphoreType.DMA,
            pltpu.SemaphoreType.DMA,
        ],
    )
    def repack(et_hbm, tabp_hbm, vbuf, obuf, vtail, otail, gsem, wsem):
        w = lax.axis_index("s") * info.num_cores + lax.axis_index("c")
        iota = lax.iota(jnp.int32, 16)
        nblk = (n_full - w + nw - 1) // nw

        def rd(c, slot):
            return pltpu.make_async_copy(
                et_hbm.at[:, pl.ds(c * lanes, lanes)], vbuf.at[slot], gsem)

        def wr(c, slot):
            return pltpu.make_async_copy(
                obuf.at[slot], tabp_hbm.at[pl.ds(c * (lanes // 2), lanes // 2), :],
                wsem)

        def transpose(slot, vb, ob, npair):
            @plsc.parallel_loop(0, npair // 16, 1, unroll=1)
            def _(pb):
                p_v = 16 * pb + iota
                col_e = 2 * p_v
                col_o = col_e + 1
                for g in range(16):
                    dk = lax.bitwise_and(g + iota, 15)
                    for kk in range(2 * d // 16):
                        k_v = 16 * kk + dk
                        row_v = lax.bitwise_and(k_v, d - 1)
                        col_v = col_o if 16 * kk >= d else col_e
                        vals = plsc.load_gather(vb, [row_v, col_v])
                        plsc.store_scatter(ob, [p_v, k_v], vals)

        @pl.when(nblk > 0)
        def _():
            rd(w, 0).start()

            def body(j, carry):
                c = w + j * nw
                slot = lax.rem(j, 2)

                @pl.when(slot == 0)
                def _():
                    rd(c, 0).wait()

                    @pl.when(j >= 2)
                    def _():
                        wr(c - 2 * nw, 0).wait()

                    transpose(0, vbuf.at[0], obuf.at[0], d)
                    wr(c, 0).start()

                    @pl.when(j + 1 < nblk)
                    def _():
                        rd(c + nw, 1).start()

                @pl.when(slot == 1)
                def _():
                    rd(c, 1).wait()

                    @pl.when(j >= 2)
                    def _():
                        wr(c - 2 * nw, 1).wait()

                    transpose(1, vbuf.at[1], obuf.at[1], d)
                    wr(c, 1).start()

                    @pl.when(j + 1 < nblk)
                    def _():
                        rd(c + nw, 0).start()

                return carry

            lax.fori_loop(0, nblk, body, 0)
            @pl.when(nblk >= 2)
            def _():
                wr(0, 0).wait()
            wr(0, 0).wait()

        @pl.when(w == nw - 1)
        def _():
            pltpu.sync_copy(et_hbm.at[:, pl.ds(n_full * lanes, rem)], vtail)

            @plsc.parallel_loop(0, rem // 32, 1, unroll=1)
            def _(pb):
                p_v = 16 * pb + iota
                col_e = 2 * p_v
                col_o = col_e + 1
                for g in range(16):
                    dk = lax.bitwise_and(g + iota, 15)
                    for kk in range(2 * d // 16):
                        k_v = 16 * kk + dk
                        row_v = lax.bitwise_and(k_v, d - 1)
                        col_v = col_o if 16 * kk >= d else col_e
                        vals = plsc.load_gather(vtail, [row_v, col_v])
                        plsc.store_scatter(otail, [p_v, k_v], vals)
            pltpu.sync_copy(
                otail, tabp_hbm.at[pl.ds(n_full * (lanes // 2), rem // 2), :])

    return repack


@functools.lru_cache(maxsize=None)
def _make_lookup(t_dim, b_dim, vh, d):
    info = plsc.get_sparse_core_info()
    nw = info.num_cores * info.num_subcores
    lanes = 128
    assert b_dim == nw * lanes
    d2 = 2 * d

    mesh = plsc.VectorSubcoreMesh(core_axis_name="c", subcore_axis_name="s")

    @functools.partial(
        pl.kernel,
        mesh=mesh,
        out_type=jax.ShapeDtypeStruct((t_dim, d, b_dim), jnp.float32),
        compiler_params=_PARAMS,
        scratch_types=[
            pltpu.VMEM((t_dim, lanes), jnp.int32),
            pltpu.VMEM((2, lanes), jnp.int32),
            pltpu.VMEM((2, lanes, d2), jnp.float32),
            pltpu.VMEM((2, d, lanes), jnp.float32),
            pltpu.SemaphoreType.DMA,
            pltpu.SemaphoreType.DMA,
        ],
    )
    def lookup(xt_hbm, tab_hbm, out_hbm, xtcol, hbuf, pairbuf, staging,
               gsem, wsem):
        w = lax.axis_index("s") * info.num_cores + lax.axis_index("c")
        col = w * lanes

        pltpu.sync_copy(xt_hbm.at[:, pl.ds(col, lanes)], xtcol)

        iota = lax.iota(jnp.int32, 16)

        def make_h(t, slot):
            for k in range(lanes // 16):
                v = xtcol[t, pl.ds(16 * k, 16)]
                hbuf[slot, pl.ds(16 * k, 16)] = lax.shift_right_logical(v, 1)

        def gather(slot):
            return pltpu.make_async_copy(
                tab_hbm.at[hbuf.at[slot]], pairbuf.at[slot], gsem)

        def wb(t, slot):
            return pltpu.make_async_copy(
                staging.at[slot], out_hbm.at[t, :, pl.ds(col, lanes)], wsem)

        def transpose(t, slot):
            @plsc.parallel_loop(0, lanes // 16, 1, unroll=1)
            def _(kb):
                bb_v = 16 * kb + iota
                idx_v = xtcol[t, pl.ds(16 * kb, 16)]
                off_v = lax.shift_left(
                    lax.bitwise_and(idx_v, jnp.int32(1)), 6)
                for g in range(16):
                    dk = lax.bitwise_and(g + iota, 15)
                    for kd in range(d // 16):
                        dd_v = 16 * kd + dk
                        vals = plsc.load_gather(
                            pairbuf.at[slot], [bb_v, off_v + dd_v])
                        plsc.store_scatter(
                            staging.at[slot], [dd_v, bb_v], vals)

        make_h(0, 0)
        gather(0).start()
        make_h(1, 1)
        gather(1).start()

        def body(i, carry):
            for par in range(2):
                t = 2 * i + par
                gather(par).wait()

                @pl.when(t >= 2)
                def _():
                    wb(t - 2, par).wait()

                transpose(t, par)
                wb(t, par).start()

                @pl.when(t + 2 < t_dim)
                def _():
                    make_h(t + 2, par)
                    gather(par).start()
            return carry

        lax.fori_loop(0, t_dim // 2, body, 0)
        wb(t_dim - 2, 0).wait()
        wb(t_dim - 1, 1).wait()

    return lookup


def kernel(x, embedding):
    b, t = x.shape
    v, d = embedding.shape
    xt = x.astype(jnp.int32).T
    et = embedding.T
    tabp = _make_repack(d, v)(et)
    out_p = _make_lookup(t, b, v // 2, d)(xt, tabp)
    return jnp.transpose(out_p, (2, 0, 1))


# speedup vs baseline: 2.5193x; 1.3334x over previous
import functools

import jax
import jax.numpy as jnp
from jax import lax
from jax.experimental import pallas as pl
from jax.experimental.pallas import tpu as pltpu
from jax.experimental.pallas import tpu_sc as plsc


@functools.lru_cache(maxsize=None)
def _make_lookup(n, v, d):
    info = plsc.get_sparse_core_info()
    nw = info.num_cores * info.num_subcores
    n_per_w = n // nw
    chunk = 256
    nbuf = 4
    n_chunks = n_per_w // chunk
    n_outer = n_chunks // nbuf

    mesh = plsc.VectorSubcoreMesh(core_axis_name="c", subcore_axis_name="s")

    @functools.partial(
        pl.kernel,
        mesh=mesh,
        out_type=jax.ShapeDtypeStruct((n, d), jnp.float32),
        compiler_params=pltpu.CompilerParams(use_tc_tiling_on_sc=False),
        scratch_types=[
            pltpu.VMEM((n_per_w,), jnp.int32),
            pltpu.VMEM((nbuf, chunk, d), jnp.float32),
            pltpu.SemaphoreType.DMA,
            pltpu.SemaphoreType.DMA,
        ],
    )
    def lookup(idx_hbm, table_hbm, out_hbm, idx_v, rows_v, gsem, wsem):
        wid = lax.axis_index("s") * info.num_cores + lax.axis_index("c")
        base = wid * n_per_w

        pltpu.sync_copy(idx_hbm.at[pl.ds(base, n_per_w)], idx_v)

        def gather_copy(g, b):
            return pltpu.make_async_copy(
                table_hbm.at[idx_v.at[pl.ds(g * chunk, chunk)]],
                rows_v.at[b], gsem)

        def wb_copy(g, b):
            return pltpu.make_async_copy(
                rows_v.at[b], out_hbm.at[pl.ds(base + g * chunk, chunk)],
                wsem)

        for b in range(nbuf):
            gather_copy(b, b).start()

        def outer(k, carry):
            for b in range(nbuf):
                g = k * nbuf + b
                gather_copy(g, b).wait()
                wb_copy(g, b).start()
                wb_copy(g, b).wait()

                @pl.when(g + nbuf < n_chunks)
                def _():
                    gather_copy(g + nbuf, b).start()
            return carry

        lax.fori_loop(0, n_outer, outer, 0)

    return lookup


def kernel(x, embedding):
    b, t = x.shape
    flat_x = x.reshape(-1).astype(jnp.int32)
    out = _make_lookup(b * t, embedding.shape[0], embedding.shape[1])(
        flat_x, embedding)
    return out.reshape(b, t, embedding.shape[1])
